# baseline (speedup 1.0000x reference)
import functools

import numpy as np
import jax
import jax.numpy as jnp
from jax import lax
from jax.experimental import pallas as pl
from jax.experimental.pallas import tpu as pltpu
from jax.experimental.pallas import tpu_sc as plsc

DT = 0.001
DZ = 10.0
NZ = 70
VMIN = 1200.0
VMAX = 6000.0
LAM = 10.0
EPS = 1e-06

NT = 1024
NXB = 256

NW = 32
NL = 16
NZP = 80
NGRP = NZP // NL
CH = 16


def _build_mats():
    n, lam = NT, LAM
    m = np.zeros((n, n), dtype=np.float64)
    i = np.arange(n)
    m[i, i] = 1.0 + 2.0 * lam
    m[0, 0] = 1.0 + lam
    m[n - 1, n - 1] = 1.0 + lam
    m[i[1:], i[:-1]] = -lam
    m[i[:-1], i[1:]] = -lam
    a = np.linalg.inv(m)
    c = 0.5 * DT * np.cumsum(a, axis=0)
    ch = c.astype(np.float32).astype(jnp.bfloat16)
    cl = (c.astype(np.float32) - np.asarray(ch, np.float32)).astype(jnp.bfloat16)
    return np.asarray(a, np.float32), ch, cl


_A_NP, _CH_NP, _CL_NP = _build_mats()


def _bsearch_bounds():
    los, his, iters = [], [], []
    for g in range(NGRP):
        ks = np.minimum(np.arange(g * NL, (g + 1) * NL), NZ - 1)
        q = ks * 10
        lo = np.maximum(-1, (q * 100) // 302 - 2).astype(np.int32)
        hi = np.minimum(NT, (q * 100) // 58 + 3).astype(np.int32)
        width = int(np.max(hi - lo))
        it = 0
        while (1 << it) < width:
            it += 1
        los.append(lo)
        his.append(hi)
        iters.append(it)
    return los, his, iters


_BS_LO, _BS_HI, _BS_ITERS = _bsearch_bounds()


def _tc_body(rms_ref, a_ref, ch_ref, cl_ref, sm_ref, zvt_ref):
    v = rms_ref[0, 0]
    t = lax.broadcasted_iota(jnp.int32, (NT, NXB), 0).astype(jnp.float32) * DT
    y = v * v * t
    dy = y[1:, :] - y[:-1, :]
    vint2_tail = dy / DT
    v0sq = jnp.clip(v[0:1, :] * v[0:1, :], VMIN * VMIN, VMAX * VMAX)
    vint2 = jnp.concatenate([v0sq, vint2_tail], axis=0)
    vint2 = jnp.clip(vint2, VMIN * VMIN, VMAX * VMAX)
    tv = jnp.sqrt(vint2 + EPS)

    sm = jnp.dot(a_ref[...], tv, preferred_element_type=jnp.float32)
    sm_ref[0, 0] = sm
    zvt_ref[0, :, NT:] = sm.T
    th = tv.astype(jnp.bfloat16)
    tl = (tv - th.astype(jnp.float32)).astype(jnp.bfloat16)
    zcol = (jnp.dot(ch_ref[...], th, preferred_element_type=jnp.float32)
            + jnp.dot(ch_ref[...], tl, preferred_element_type=jnp.float32)
            + jnp.dot(cl_ref[...], th, preferred_element_type=jnp.float32))
    zvt_ref[0, :, :NT] = zcol.T


def _tc_call(rms_vel, a_mat, ch_mat, cl_mat):
    b, _, nt, nx = rms_vel.shape
    return pl.pallas_call(
        _tc_body,
        grid=(b, nx // NXB),
        in_specs=[
            pl.BlockSpec((1, 1, NT, NXB), lambda i, j: (i, 0, 0, j)),
            pl.BlockSpec((NT, NT), lambda i, j: (0, 0)),
            pl.BlockSpec((NT, NT), lambda i, j: (0, 0)),
            pl.BlockSpec((NT, NT), lambda i, j: (0, 0)),
        ],
        out_specs=[
            pl.BlockSpec((1, 1, NT, NXB), lambda i, j: (i, 0, 0, j)),
            pl.BlockSpec((1, NXB, 2 * NT), lambda i, j: (i, j, 0)),
        ],
        out_shape=[
            jax.ShapeDtypeStruct((b, 1, nt, nx), jnp.float32),
            jax.ShapeDtypeStruct((b, nx, 2 * nt), jnp.float32),
        ],
    )(rms_vel, a_mat, ch_mat, cl_mat)


def _search_chunk(zv_buf, o_buf, out_hbm, row):
    ti = lax.iota(jnp.int32, NL)

    def k_body(k, carry):
        qf = k.astype(jnp.float32) * DZ
        lo0 = jnp.maximum((k * 3390) >> 10, 1) - 2
        lo = jnp.broadcast_to(lo0, (NL,))
        for h in (512, 256, 128, 64, 32, 16, 8, 4, 2, 1):
            mid = lo + h
            zm = plsc.load_gather(zv_buf, [ti, mid])
            lo = jnp.where(zm < qf, mid, lo)
        idx1 = jnp.minimum(lo + 1, NT - 1)
        idx0 = jnp.clip(lo, 0, NT - 2)
        z0 = plsc.load_gather(zv_buf, [ti, idx0])
        z1 = plsc.load_gather(zv_buf, [ti, idx1])
        v0 = plsc.load_gather(zv_buf, [ti, idx0 + NT])
        v1 = plsc.load_gather(zv_buf, [ti, idx1 + NT])
        denom = jnp.maximum(z1 - z0, EPS)
        w = jnp.clip((qf - z0) / denom, 0.0, 1.0)
        plsc.store_scatter(o_buf, [ti, jnp.broadcast_to(k, (NL,))],
                           v0 + w * (v1 - v0))
        return carry

    lax.fori_loop(0, NZ, k_body, 0)
    pltpu.sync_copy(o_buf, out_hbm.at[pl.ds(row, CH)])


def _sc_body(zv_hbm, out_hbm, zv0, zv1, o_buf, sem0, sem1):
    ntr = zv_hbm.shape[0]
    per_w = ntr // NW
    nch = per_w // CH
    wid = lax.axis_index("s") * 2 + lax.axis_index("c")
    base = wid * per_w

    def issue(row, buf, sem):
        pltpu.async_copy(zv_hbm.at[pl.ds(row, CH)], buf, sem)

    def drain(buf, sem):
        pltpu.make_async_copy(zv_hbm.at[pl.ds(0, CH)], buf, sem).wait()

    issue(base, zv0, sem0)

    def pair_body(i, carry):
        r0 = base + (2 * i) * CH
        r1 = r0 + CH
        issue(r1, zv1, sem1)
        drain(zv0, sem0)
        _search_chunk(zv0, o_buf, out_hbm, r0)

        @pl.when(i < (nch // 2) - 1)
        def _():
            issue(r1 + CH, zv0, sem0)

        drain(zv1, sem1)
        _search_chunk(zv1, o_buf, out_hbm, r1)
        return carry

    lax.fori_loop(0, nch // 2, pair_body, 0)


def _depth_resample(zvflat):
    ntr = zvflat.shape[0]
    mesh = plsc.VectorSubcoreMesh(core_axis_name="c", subcore_axis_name="s")
    fn = pl.kernel(
        _sc_body,
        mesh=mesh,
        out_type=jax.ShapeDtypeStruct((ntr, NZP), jnp.float32),
        scratch_types=[
            pltpu.VMEM((CH, 2 * NT), jnp.float32),
            pltpu.VMEM((CH, 2 * NT), jnp.float32),
            pltpu.VMEM((CH, NZP), jnp.float32),
            pltpu.SemaphoreType.DMA,
            pltpu.SemaphoreType.DMA,
        ],
        compiler_params=pltpu.CompilerParams(needs_layout_passes=False),
    )
    return fn(zvflat)


def kernel(rms_vel):
    b, _, nt, nx = rms_vel.shape
    a_mat = jnp.asarray(_A_NP)
    ch_mat = jnp.asarray(_CH_NP)
    cl_mat = jnp.asarray(_CL_NP)
    sm, zvt = _tc_call(rms_vel, a_mat, ch_mat, cl_mat)
    depth_flat = _depth_resample(zvt.reshape(b * nx, 2 * nt))
    depth = depth_flat[:, :NZ].reshape(b, nx, NZ)
    depth_vel = jnp.transpose(depth, (0, 2, 1))[:, None]
    return depth_vel, sm


# speedup vs baseline: 1.5338x; 1.5338x over previous
import functools

import numpy as np
import jax
import jax.numpy as jnp
from jax import lax
from jax.experimental import pallas as pl
from jax.experimental.pallas import tpu as pltpu
from jax.experimental.pallas import tpu_sc as plsc

DT = 0.001
DZ = 10.0
NZ = 70
VMIN = 1200.0
VMAX = 6000.0
LAM = 10.0
EPS = 1e-06

NT = 1024
NXB = 256

NW = 32
NL = 16
NZP = 80
NGRP = NZP // NL
CH = 16


def _build_mats():
    n, lam = NT, LAM
    m = np.zeros((n, n), dtype=np.float64)
    i = np.arange(n)
    m[i, i] = 1.0 + 2.0 * lam
    m[0, 0] = 1.0 + lam
    m[n - 1, n - 1] = 1.0 + lam
    m[i[1:], i[:-1]] = -lam
    m[i[:-1], i[1:]] = -lam
    a = np.linalg.inv(m)
    c = 0.5 * DT * np.cumsum(a, axis=0)
    ch = c.astype(np.float32).astype(jnp.bfloat16)
    cl = (c.astype(np.float32) - np.asarray(ch, np.float32)).astype(jnp.bfloat16)
    return np.asarray(a, np.float32), ch, cl


_A_NP, _CH_NP, _CL_NP = _build_mats()


def _bsearch_bounds():
    los, his, iters = [], [], []
    for g in range(NGRP):
        ks = np.minimum(np.arange(g * NL, (g + 1) * NL), NZ - 1)
        q = ks * 10
        lo = np.maximum(-1, (q * 100) // 302 - 2).astype(np.int32)
        hi = np.minimum(NT, (q * 100) // 58 + 3).astype(np.int32)
        width = int(np.max(hi - lo))
        it = 0
        while (1 << it) < width:
            it += 1
        los.append(lo)
        his.append(hi)
        iters.append(it)
    return los, his, iters


_BS_LO, _BS_HI, _BS_ITERS = _bsearch_bounds()


def _tc_body(rms_ref, a_ref, ch_ref, cl_ref, sm_ref, zvt_ref):
    v = rms_ref[0, 0]
    t = lax.broadcasted_iota(jnp.int32, (NT, NXB), 0).astype(jnp.float32) * DT
    y = v * v * t
    dy = y[1:, :] - y[:-1, :]
    vint2_tail = dy / DT
    v0sq = jnp.clip(v[0:1, :] * v[0:1, :], VMIN * VMIN, VMAX * VMAX)
    vint2 = jnp.concatenate([v0sq, vint2_tail], axis=0)
    vint2 = jnp.clip(vint2, VMIN * VMIN, VMAX * VMAX)
    tv = jnp.sqrt(vint2 + EPS)

    sm = jnp.dot(a_ref[...], tv, preferred_element_type=jnp.float32)
    sm_ref[0, 0] = sm
    zvt_ref[0, :, NT:] = sm.T
    th = tv.astype(jnp.bfloat16)
    tl = (tv - th.astype(jnp.float32)).astype(jnp.bfloat16)
    zcol = (jnp.dot(ch_ref[...], th, preferred_element_type=jnp.float32)
            + jnp.dot(ch_ref[...], tl, preferred_element_type=jnp.float32)
            + jnp.dot(cl_ref[...], th, preferred_element_type=jnp.float32))
    zvt_ref[0, :, :NT] = zcol.T


def _tc_call(rms_vel, a_mat, ch_mat, cl_mat):
    b, _, nt, nx = rms_vel.shape
    return pl.pallas_call(
        _tc_body,
        grid=(b, nx // NXB),
        in_specs=[
            pl.BlockSpec((1, 1, NT, NXB), lambda i, j: (i, 0, 0, j)),
            pl.BlockSpec((NT, NT), lambda i, j: (0, 0)),
            pl.BlockSpec((NT, NT), lambda i, j: (0, 0)),
            pl.BlockSpec((NT, NT), lambda i, j: (0, 0)),
        ],
        out_specs=[
            pl.BlockSpec((1, 1, NT, NXB), lambda i, j: (i, 0, 0, j)),
            pl.BlockSpec((1, NXB, 2 * NT), lambda i, j: (i, j, 0)),
        ],
        out_shape=[
            jax.ShapeDtypeStruct((b, 1, nt, nx), jnp.float32),
            jax.ShapeDtypeStruct((b, nx, 2 * nt), jnp.float32),
        ],
    )(rms_vel, a_mat, ch_mat, cl_mat)


def _search_chunk(zv_buf, o_buf, out_hbm, row):
    ti = lax.iota(jnp.int32, NL)
    kw = 8

    def k_body(i, carry):
        k8 = i * kw
        los, qfs = [], []
        for j in range(kw):
            kq = jnp.minimum(k8 + j, NZ - 1)
            qfs.append(kq.astype(jnp.float32) * DZ)
            lo0 = jnp.maximum((kq * 3390) >> 10, 1) - 2
            los.append(jnp.broadcast_to(lo0, (NL,)))
        for h in (512, 256, 128, 64, 32, 16, 8, 4, 2, 1):
            zms = [plsc.load_gather(zv_buf, [ti, los[j] + h])
                   for j in range(kw)]
            los = [jnp.where(zms[j] < qfs[j], los[j] + h, los[j])
                   for j in range(kw)]
        for j in range(kw):
            lo = los[j]
            qf = qfs[j]
            idx1 = jnp.minimum(lo + 1, NT - 1)
            idx0 = jnp.clip(lo, 0, NT - 2)
            z0 = plsc.load_gather(zv_buf, [ti, idx0])
            z1 = plsc.load_gather(zv_buf, [ti, idx1])
            v0 = plsc.load_gather(zv_buf, [ti, idx0 + NT])
            v1 = plsc.load_gather(zv_buf, [ti, idx1 + NT])
            denom = jnp.maximum(z1 - z0, EPS)
            w = jnp.clip((qf - z0) / denom, 0.0, 1.0)
            plsc.store_scatter(o_buf, [ti, jnp.broadcast_to(k8 + j, (NL,))],
                               v0 + w * (v1 - v0))
        return carry

    lax.fori_loop(0, (NZ + kw - 1) // kw, k_body, 0)
    pltpu.sync_copy(o_buf, out_hbm.at[pl.ds(row, CH)])


def _sc_body(zv_hbm, out_hbm, zv0, zv1, o_buf, sem0, sem1):
    ntr = zv_hbm.shape[0]
    per_w = ntr // NW
    nch = per_w // CH
    wid = lax.axis_index("s") * 2 + lax.axis_index("c")
    base = wid * per_w

    def issue(row, buf, sem):
        pltpu.async_copy(zv_hbm.at[pl.ds(row, CH)], buf, sem)

    def drain(buf, sem):
        pltpu.make_async_copy(zv_hbm.at[pl.ds(0, CH)], buf, sem).wait()

    issue(base, zv0, sem0)

    def pair_body(i, carry):
        r0 = base + (2 * i) * CH
        r1 = r0 + CH
        issue(r1, zv1, sem1)
        drain(zv0, sem0)
        _search_chunk(zv0, o_buf, out_hbm, r0)

        @pl.when(i < (nch // 2) - 1)
        def _():
            issue(r1 + CH, zv0, sem0)

        drain(zv1, sem1)
        _search_chunk(zv1, o_buf, out_hbm, r1)
        return carry

    lax.fori_loop(0, nch // 2, pair_body, 0)


def _depth_resample(zvflat):
    ntr = zvflat.shape[0]
    mesh = plsc.VectorSubcoreMesh(core_axis_name="c", subcore_axis_name="s")
    fn = pl.kernel(
        _sc_body,
        mesh=mesh,
        out_type=jax.ShapeDtypeStruct((ntr, NZP), jnp.float32),
        scratch_types=[
            pltpu.VMEM((CH, 2 * NT), jnp.float32),
            pltpu.VMEM((CH, 2 * NT), jnp.float32),
            pltpu.VMEM((CH, NZP), jnp.float32),
            pltpu.SemaphoreType.DMA,
            pltpu.SemaphoreType.DMA,
        ],
        compiler_params=pltpu.CompilerParams(needs_layout_passes=False),
    )
    return fn(zvflat)


def kernel(rms_vel):
    b, _, nt, nx = rms_vel.shape
    a_mat = jnp.asarray(_A_NP)
    ch_mat = jnp.asarray(_CH_NP)
    cl_mat = jnp.asarray(_CL_NP)
    sm, zvt = _tc_call(rms_vel, a_mat, ch_mat, cl_mat)
    depth_flat = _depth_resample(zvt.reshape(b * nx, 2 * nt))
    depth = depth_flat[:, :NZ].reshape(b, nx, NZ)
    depth_vel = jnp.transpose(depth, (0, 2, 1))[:, None]
    return depth_vel, sm


# speedup vs baseline: 1.5630x; 1.0190x over previous
import functools

import numpy as np
import jax
import jax.numpy as jnp
from jax import lax
from jax.experimental import pallas as pl
from jax.experimental.pallas import tpu as pltpu
from jax.experimental.pallas import tpu_sc as plsc

DT = 0.001
DZ = 10.0
NZ = 70
VMIN = 1200.0
VMAX = 6000.0
LAM = 10.0
EPS = 1e-06

NT = 1024
NXB = 256

NW = 32
NL = 16
NZP = 80
NGRP = NZP // NL
CH = 16


def _build_mats():
    n, lam = NT, LAM
    m = np.zeros((n, n), dtype=np.float64)
    i = np.arange(n)
    m[i, i] = 1.0 + 2.0 * lam
    m[0, 0] = 1.0 + lam
    m[n - 1, n - 1] = 1.0 + lam
    m[i[1:], i[:-1]] = -lam
    m[i[:-1], i[1:]] = -lam
    a = np.linalg.inv(m)
    c = 0.5 * DT * np.cumsum(a, axis=0)
    ch = c.astype(np.float32).astype(jnp.bfloat16)
    cl = (c.astype(np.float32) - np.asarray(ch, np.float32)).astype(jnp.bfloat16)
    return np.asarray(a, np.float32), ch, cl


_A_NP, _CH_NP, _CL_NP = _build_mats()


def _bsearch_bounds():
    los, his, iters = [], [], []
    for g in range(NGRP):
        ks = np.minimum(np.arange(g * NL, (g + 1) * NL), NZ - 1)
        q = ks * 10
        lo = np.maximum(-1, (q * 100) // 302 - 2).astype(np.int32)
        hi = np.minimum(NT, (q * 100) // 58 + 3).astype(np.int32)
        width = int(np.max(hi - lo))
        it = 0
        while (1 << it) < width:
            it += 1
        los.append(lo)
        his.append(hi)
        iters.append(it)
    return los, his, iters


_BS_LO, _BS_HI, _BS_ITERS = _bsearch_bounds()


def _tc_body(rms_ref, a_ref, ch_ref, cl_ref, sm_ref, zvt_ref):
    v = rms_ref[0, 0]
    t = lax.broadcasted_iota(jnp.int32, (NT, NXB), 0).astype(jnp.float32) * DT
    y = v * v * t
    dy = y[1:, :] - y[:-1, :]
    vint2_tail = dy / DT
    v0sq = jnp.clip(v[0:1, :] * v[0:1, :], VMIN * VMIN, VMAX * VMAX)
    vint2 = jnp.concatenate([v0sq, vint2_tail], axis=0)
    vint2 = jnp.clip(vint2, VMIN * VMIN, VMAX * VMAX)
    tv = jnp.sqrt(vint2 + EPS)

    sm = jnp.dot(a_ref[...], tv, preferred_element_type=jnp.float32)
    sm_ref[0, 0] = sm
    zvt_ref[0, :, NT:] = sm.T
    th = tv.astype(jnp.bfloat16)
    tl = (tv - th.astype(jnp.float32)).astype(jnp.bfloat16)
    zcol = (jnp.dot(ch_ref[...], th, preferred_element_type=jnp.float32)
            + jnp.dot(ch_ref[...], tl, preferred_element_type=jnp.float32)
            + jnp.dot(cl_ref[...], th, preferred_element_type=jnp.float32))
    zvt_ref[0, :, :NT] = zcol.T


def _tc_call(rms_vel, a_mat, ch_mat, cl_mat):
    b, _, nt, nx = rms_vel.shape
    return pl.pallas_call(
        _tc_body,
        grid=(b, nx // NXB),
        in_specs=[
            pl.BlockSpec((1, 1, NT, NXB), lambda i, j: (i, 0, 0, j)),
            pl.BlockSpec((NT, NT), lambda i, j: (0, 0)),
            pl.BlockSpec((NT, NT), lambda i, j: (0, 0)),
            pl.BlockSpec((NT, NT), lambda i, j: (0, 0)),
        ],
        out_specs=[
            pl.BlockSpec((1, 1, NT, NXB), lambda i, j: (i, 0, 0, j)),
            pl.BlockSpec((1, NXB, 2 * NT), lambda i, j: (i, j, 0)),
        ],
        out_shape=[
            jax.ShapeDtypeStruct((b, 1, nt, nx), jnp.float32),
            jax.ShapeDtypeStruct((b, nx, 2 * nt), jnp.float32),
        ],
    )(rms_vel, a_mat, ch_mat, cl_mat)


def _search_chunk(zv_buf, o_buf, out_hbm, row):
    ti = lax.iota(jnp.int32, NL)
    kw = 8

    def k_body(i, carry):
        k8 = i * kw
        los, qfs = [], []
        for j in range(kw):
            kq = jnp.minimum(k8 + j, NZ - 1)
            qfs.append(kq.astype(jnp.float32) * DZ)
            lo0 = jnp.maximum((kq * 3390) >> 10, 1) - 2
            los.append(jnp.broadcast_to(lo0, (NL,)))
        for h in (512, 256, 128, 64, 32, 16, 8, 4, 2, 1):
            zms = [plsc.load_gather(zv_buf, [ti, los[j] + h])
                   for j in range(kw)]
            los = [jnp.where(zms[j] < qfs[j], los[j] + h, los[j])
                   for j in range(kw)]
        for j in range(kw):
            lo = los[j]
            qf = qfs[j]
            idx1 = jnp.minimum(lo + 1, NT - 1)
            idx0 = jnp.clip(lo, 0, NT - 2)
            z0 = plsc.load_gather(zv_buf, [ti, idx0])
            z1 = plsc.load_gather(zv_buf, [ti, idx1])
            v0 = plsc.load_gather(zv_buf, [ti, idx0 + NT])
            v1 = plsc.load_gather(zv_buf, [ti, idx1 + NT])
            denom = jnp.maximum(z1 - z0, EPS)
            w = jnp.clip((qf - z0) / denom, 0.0, 1.0)
            plsc.store_scatter(o_buf, [ti, jnp.broadcast_to(k8 + j, (NL,))],
                               v0 + w * (v1 - v0))
        return carry

    lax.fori_loop(0, (NZ + kw - 1) // kw, k_body, 0)
    pltpu.sync_copy(o_buf, out_hbm.at[pl.ds(row, CH)])


def _sc_body(zv_hbm, out_hbm, zv0, zv1, o_buf, sem0, sem1):
    ntr = zv_hbm.shape[0]
    per_w = ntr // NW
    nch = per_w // CH
    wid = lax.axis_index("s") * 2 + lax.axis_index("c")
    base = wid * per_w

    def issue(row, buf, sem):
        pltpu.async_copy(zv_hbm.at[pl.ds(row, CH)], buf, sem)

    def drain(buf, sem):
        pltpu.make_async_copy(zv_hbm.at[pl.ds(0, CH)], buf, sem).wait()

    issue(base, zv0, sem0)

    def pair_body(i, carry):
        r0 = base + (2 * i) * CH
        r1 = r0 + CH
        issue(r1, zv1, sem1)
        drain(zv0, sem0)
        _search_chunk(zv0, o_buf, out_hbm, r0)

        @pl.when(i < (nch // 2) - 1)
        def _():
            issue(r1 + CH, zv0, sem0)

        drain(zv1, sem1)
        _search_chunk(zv1, o_buf, out_hbm, r1)
        return carry

    lax.fori_loop(0, nch // 2, pair_body, 0)


def _depth_resample(zvflat):
    ntr = zvflat.shape[0]
    mesh = plsc.VectorSubcoreMesh(core_axis_name="c", subcore_axis_name="s")
    fn = pl.kernel(
        _sc_body,
        mesh=mesh,
        out_type=jax.ShapeDtypeStruct((ntr, NZP), jnp.float32),
        scratch_types=[
            pltpu.VMEM((CH, 2 * NT), jnp.float32),
            pltpu.VMEM((CH, 2 * NT), jnp.float32),
            pltpu.VMEM((CH, NZP), jnp.float32),
            pltpu.SemaphoreType.DMA,
            pltpu.SemaphoreType.DMA,
        ],
        compiler_params=pltpu.CompilerParams(needs_layout_passes=False),
    )
    return fn(zvflat)


def kernel(rms_vel):
    b, _, nt, nx = rms_vel.shape
    a_mat = jnp.asarray(_A_NP)
    ch_mat = jnp.asarray(_CH_NP)
    cl_mat = jnp.asarray(_CL_NP)
    bh = b // 2
    sms, depths = [], []
    for h in range(2):
        part = lax.slice_in_dim(rms_vel, h * bh, (h + 1) * bh, axis=0)
        sm, zvt = _tc_call(part, a_mat, ch_mat, cl_mat)
        depth_flat = _depth_resample(zvt.reshape(bh * nx, 2 * nt))
        sms.append(sm)
        depths.append(depth_flat[:, :NZ].reshape(bh, nx, NZ))
    depth = jnp.concatenate(depths, axis=0)
    depth_vel = jnp.transpose(depth, (0, 2, 1))[:, None]
    return depth_vel, jnp.concatenate(sms, axis=0)


# speedup vs baseline: 1.6301x; 1.0430x over previous
import functools

import numpy as np
import jax
import jax.numpy as jnp
from jax import lax
from jax.experimental import pallas as pl
from jax.experimental.pallas import tpu as pltpu
from jax.experimental.pallas import tpu_sc as plsc

DT = 0.001
DZ = 10.0
NZ = 70
VMIN = 1200.0
VMAX = 6000.0
LAM = 10.0
EPS = 1e-06

NT = 1024
NXB = 256

NW = 32
NL = 16
NZP = 80
NGRP = NZP // NL
CH = 16


def _build_mats():
    n, lam = NT, LAM
    m = np.zeros((n, n), dtype=np.float64)
    i = np.arange(n)
    m[i, i] = 1.0 + 2.0 * lam
    m[0, 0] = 1.0 + lam
    m[n - 1, n - 1] = 1.0 + lam
    m[i[1:], i[:-1]] = -lam
    m[i[:-1], i[1:]] = -lam
    a = np.linalg.inv(m)
    c = 0.5 * DT * np.cumsum(a, axis=0)
    ch = c.astype(np.float32).astype(jnp.bfloat16)
    cl = (c.astype(np.float32) - np.asarray(ch, np.float32)).astype(jnp.bfloat16)
    return np.asarray(a, np.float32), ch, cl


_A_NP, _CH_NP, _CL_NP = _build_mats()


def _bsearch_bounds():
    los, his, iters = [], [], []
    for g in range(NGRP):
        ks = np.minimum(np.arange(g * NL, (g + 1) * NL), NZ - 1)
        q = ks * 10
        lo = np.maximum(-1, (q * 100) // 302 - 2).astype(np.int32)
        hi = np.minimum(NT, (q * 100) // 58 + 3).astype(np.int32)
        width = int(np.max(hi - lo))
        it = 0
        while (1 << it) < width:
            it += 1
        los.append(lo)
        his.append(hi)
        iters.append(it)
    return los, his, iters


_BS_LO, _BS_HI, _BS_ITERS = _bsearch_bounds()


def _tc_body(rms_ref, a_ref, ch_ref, cl_ref, sm_ref, zvt_ref):
    v = rms_ref[0, 0]
    t = lax.broadcasted_iota(jnp.int32, (NT, NXB), 0).astype(jnp.float32) * DT
    y = v * v * t
    dy = y[1:, :] - y[:-1, :]
    vint2_tail = dy / DT
    v0sq = jnp.clip(v[0:1, :] * v[0:1, :], VMIN * VMIN, VMAX * VMAX)
    vint2 = jnp.concatenate([v0sq, vint2_tail], axis=0)
    vint2 = jnp.clip(vint2, VMIN * VMIN, VMAX * VMAX)
    tv = jnp.sqrt(vint2 + EPS)

    sm = jnp.dot(a_ref[...], tv, preferred_element_type=jnp.float32)
    sm_ref[0, 0] = sm
    zvt_ref[0, :, NT:] = sm.T
    th = tv.astype(jnp.bfloat16)
    tl = (tv - th.astype(jnp.float32)).astype(jnp.bfloat16)
    zcol = (jnp.dot(ch_ref[...], th, preferred_element_type=jnp.float32)
            + jnp.dot(ch_ref[...], tl, preferred_element_type=jnp.float32)
            + jnp.dot(cl_ref[...], th, preferred_element_type=jnp.float32))
    zvt_ref[0, :, :NT] = zcol.T


def _tc_call(rms_vel, a_mat, ch_mat, cl_mat):
    b, _, nt, nx = rms_vel.shape
    return pl.pallas_call(
        _tc_body,
        grid=(b, nx // NXB),
        in_specs=[
            pl.BlockSpec((1, 1, NT, NXB), lambda i, j: (i, 0, 0, j)),
            pl.BlockSpec((NT, NT), lambda i, j: (0, 0)),
            pl.BlockSpec((NT, NT), lambda i, j: (0, 0)),
            pl.BlockSpec((NT, NT), lambda i, j: (0, 0)),
        ],
        out_specs=[
            pl.BlockSpec((1, 1, NT, NXB), lambda i, j: (i, 0, 0, j)),
            pl.BlockSpec((1, NXB, 2 * NT), lambda i, j: (i, j, 0)),
        ],
        out_shape=[
            jax.ShapeDtypeStruct((b, 1, nt, nx), jnp.float32),
            jax.ShapeDtypeStruct((b, nx, 2 * nt), jnp.float32),
        ],
    )(rms_vel, a_mat, ch_mat, cl_mat)


def _search_chunk(zv_buf, o_buf, out_hbm, row):
    ti = lax.iota(jnp.int32, NL)
    nchain = 8
    kpc = 9

    def probe(lo, h, qf):
        mid = lo + h
        zm = plsc.load_gather(zv_buf, [ti, mid])
        return jnp.where(zm < qf, mid, lo)

    def emit(lo, qf, col):
        idx1 = jnp.minimum(lo + 1, NT - 1)
        idx0 = jnp.clip(lo, 0, NT - 2)
        z0 = plsc.load_gather(zv_buf, [ti, idx0])
        z1 = plsc.load_gather(zv_buf, [ti, idx1])
        v0 = plsc.load_gather(zv_buf, [ti, idx0 + NT])
        v1 = plsc.load_gather(zv_buf, [ti, idx1 + NT])
        denom = jnp.maximum(z1 - z0, EPS)
        w = jnp.clip((qf - z0) / denom, 0.0, 1.0)
        plsc.store_scatter(o_buf, [ti, jnp.broadcast_to(col, (NL,))],
                           v0 + w * (v1 - v0))

    def qof(k):
        return float(min(k, NZ - 1)) * DZ

    los = []
    for j in range(nchain):
        k0 = j * kpc
        lo0 = max((k0 * 3390) >> 10, 1) - 2
        los.append(jnp.full((NL,), lo0, jnp.int32))
    for h in (512, 256, 128, 64, 32, 16, 8, 4, 2, 1):
        los = [probe(los[j], h, qof(j * kpc)) for j in range(nchain)]
    for j in range(nchain):
        emit(los[j], qof(j * kpc), j * kpc)
    for pos in range(1, kpc):
        for h in (16, 8, 4, 2, 1):
            los = [probe(los[j], h, qof(j * kpc + pos))
                   for j in range(nchain)]
        for j in range(nchain):
            emit(los[j], qof(j * kpc + pos), j * kpc + pos)

    pltpu.sync_copy(o_buf, out_hbm.at[pl.ds(row, CH)])


def _sc_body(zv_hbm, out_hbm, zv0, zv1, o_buf, sem0, sem1):
    ntr = zv_hbm.shape[0]
    per_w = ntr // NW
    nch = per_w // CH
    wid = lax.axis_index("s") * 2 + lax.axis_index("c")
    base = wid * per_w

    def issue(row, buf, sem):
        pltpu.async_copy(zv_hbm.at[pl.ds(row, CH)], buf, sem)

    def drain(buf, sem):
        pltpu.make_async_copy(zv_hbm.at[pl.ds(0, CH)], buf, sem).wait()

    issue(base, zv0, sem0)

    def pair_body(i, carry):
        r0 = base + (2 * i) * CH
        r1 = r0 + CH
        issue(r1, zv1, sem1)
        drain(zv0, sem0)
        _search_chunk(zv0, o_buf, out_hbm, r0)

        @pl.when(i < (nch // 2) - 1)
        def _():
            issue(r1 + CH, zv0, sem0)

        drain(zv1, sem1)
        _search_chunk(zv1, o_buf, out_hbm, r1)
        return carry

    lax.fori_loop(0, nch // 2, pair_body, 0)


def _depth_resample(zvflat):
    ntr = zvflat.shape[0]
    mesh = plsc.VectorSubcoreMesh(core_axis_name="c", subcore_axis_name="s")
    fn = pl.kernel(
        _sc_body,
        mesh=mesh,
        out_type=jax.ShapeDtypeStruct((ntr, NZP), jnp.float32),
        scratch_types=[
            pltpu.VMEM((CH, 2 * NT), jnp.float32),
            pltpu.VMEM((CH, 2 * NT), jnp.float32),
            pltpu.VMEM((CH, NZP), jnp.float32),
            pltpu.SemaphoreType.DMA,
            pltpu.SemaphoreType.DMA,
        ],
        compiler_params=pltpu.CompilerParams(needs_layout_passes=False),
    )
    return fn(zvflat)


def kernel(rms_vel):
    b, _, nt, nx = rms_vel.shape
    a_mat = jnp.asarray(_A_NP)
    ch_mat = jnp.asarray(_CH_NP)
    cl_mat = jnp.asarray(_CL_NP)
    bh = b // 2
    sms, depths = [], []
    for h in range(2):
        part = lax.slice_in_dim(rms_vel, h * bh, (h + 1) * bh, axis=0)
        sm, zvt = _tc_call(part, a_mat, ch_mat, cl_mat)
        depth_flat = _depth_resample(zvt.reshape(bh * nx, 2 * nt))
        sms.append(sm)
        depths.append(depth_flat[:, :NZ].reshape(bh, nx, NZ))
    depth = jnp.concatenate(depths, axis=0)
    depth_vel = jnp.transpose(depth, (0, 2, 1))[:, None]
    return depth_vel, jnp.concatenate(sms, axis=0)
